# baseline (speedup 1.0000x reference)
import functools

import jax
import jax.numpy as jnp
from jax import lax
from jax.experimental import pallas as pl
from jax.experimental.pallas import tpu as pltpu
from jax.experimental.pallas import tpu_sc as plsc

N = 262144
CIN = 9
M = 40000
F = 256
COMPR = 16

_PREC = lax.Precision.HIGHEST
_EPS = 1e-5


_BX = 2048


def _lrelu(x):
    return jnp.where(x >= 0, x, 0.01 * x)


def _xstats_kernel(x_ref, s_ref, q_ref):
    i = pl.program_id(0)

    @pl.when(i == 0)
    def _():
        s_ref[...] = jnp.zeros_like(s_ref)
        q_ref[...] = jnp.zeros_like(q_ref)

    xb = x_ref[...]
    s_ref[...] += jnp.sum(xb, axis=0, keepdims=True)
    q_ref[...] += lax.dot_general(
        xb, xb, (((0,), (0,)), ((), ())),
        preferred_element_type=jnp.float32, precision=_PREC)


def _xstats(x):
    return pl.pallas_call(
        _xstats_kernel,
        grid=(N // _BX,),
        in_specs=[pl.BlockSpec((_BX, CIN), lambda i: (i, 0))],
        out_specs=[
            pl.BlockSpec((1, CIN), lambda i: (0, 0)),
            pl.BlockSpec((CIN, CIN), lambda i: (0, 0)),
        ],
        out_shape=[
            jax.ShapeDtypeStruct((1, CIN), jnp.float32),
            jax.ShapeDtypeStruct((CIN, CIN), jnp.float32),
        ],
    )(x)


def _fwd(x, ws, depth):
    h = lax.dot_general(x, ws["W1f"], (((1,), (0,)), ((), ())),
                        preferred_element_type=jnp.float32, precision=_PREC)
    h = h + ws["b1f"]
    if depth == 2:
        return h
    h = _lrelu(h * ws["s1"] + ws["t1"])
    h = lax.dot_general(h, ws["W2"], (((1,), (0,)), ((), ())),
                        preferred_element_type=jnp.float32, precision=_PREC)
    h = h + ws["b2"]
    if depth == 3:
        return h
    h = _lrelu(h * ws["s2"] + ws["t2"])
    h = lax.dot_general(h, ws["W3"], (((1,), (0,)), ((), ())),
                        preferred_element_type=jnp.float32, precision=_PREC)
    h = h + ws["b3"]
    if depth == 4:
        return h
    h = _lrelu(h * ws["s3"] + ws["t3"])
    h = lax.dot_general(h, ws["W4"], (((1,), (0,)), ((), ())),
                        preferred_element_type=jnp.float32, precision=_PREC)
    return h + ws["b4"]


_STAT_KEYS = {
    3: ("W1f", "b1f", "s1", "t1", "W2", "b2"),
    4: ("W1f", "b1f", "s1", "t1", "W2", "b2", "s2", "t2", "W3", "b3"),
    5: ("W1f", "b1f", "s1", "t1", "W2", "b2", "s2", "t2", "W3", "b3",
        "s3", "t3", "W4", "b4"),
}


def _stats_kernel(depth, keys, x_ref, *rest):
    w_refs, (o_ref,) = rest[:len(keys)], rest[len(keys):]
    i = pl.program_id(0)

    @pl.when(i == 0)
    def _():
        o_ref[...] = jnp.zeros_like(o_ref)

    ws = {k: r[...] for k, r in zip(keys, w_refs)}
    a = _fwd(x_ref[...], ws, depth)
    o_ref[0:1, :] += jnp.sum(a, axis=0, keepdims=True)
    o_ref[1:2, :] += jnp.sum(a * a, axis=0, keepdims=True)


def _run_stats(x, ws, depth, width):
    keys = _STAT_KEYS[depth]
    return pl.pallas_call(
        functools.partial(_stats_kernel, depth, keys),
        grid=(N // _BX,),
        in_specs=[pl.BlockSpec((_BX, CIN), lambda i: (i, 0))] + [
            pl.BlockSpec(ws[k].shape, lambda i: (0, 0)) for k in keys],
        out_specs=[pl.BlockSpec((2, width), lambda i: (0, 0))],
        out_shape=[jax.ShapeDtypeStruct((2, width), jnp.float32)],
    )(x, *[ws[k] for k in keys])


def _feat_kernel(keys, x_ref, *rest):
    w_refs, (o_ref,) = rest[:len(keys)], rest[len(keys):]
    ws = {k: r[...] for k, r in zip(keys, w_refs)}
    o_ref[...] = _fwd(x_ref[...], ws, 5)


def _run_feat(x, ws):
    keys = _STAT_KEYS[5]
    return pl.pallas_call(
        functools.partial(_feat_kernel, keys),
        grid=(N // _BX,),
        in_specs=[pl.BlockSpec((_BX, CIN), lambda i: (i, 0))] + [
            pl.BlockSpec(ws[k].shape, lambda i: (0, 0)) for k in keys],
        out_specs=[pl.BlockSpec((_BX, F), lambda i: (i, 0))],
        out_shape=[jax.ShapeDtypeStruct((N, F), jnp.float32)],
    )(x, *[ws[k] for k in keys])[0]


_BM = 2000


def _compress_kernel(s_ref, w_ref, b_ref, o_ref):
    h = lax.dot_general(s_ref[...], w_ref[...], (((1,), (0,)), ((), ())),
                        preferred_element_type=jnp.float32, precision=_PREC)
    o_ref[...] = _lrelu(h + b_ref[...])


def _run_compress(smax, Wc, bc):
    return pl.pallas_call(
        _compress_kernel,
        grid=(M // _BM,),
        in_specs=[
            pl.BlockSpec((_BM, F), lambda i: (i, 0)),
            pl.BlockSpec((F, COMPR), lambda i: (0, 0)),
            pl.BlockSpec((1, COMPR), lambda i: (0, 0)),
        ],
        out_specs=[pl.BlockSpec((_BM, COMPR), lambda i: (i, 0))],
        out_shape=[jax.ShapeDtypeStruct((M, COMPR), jnp.float32)],
    )(smax, Wc, bc.reshape(1, COMPR))[0]


_NC = 2
_NS = 16
_NW = _NC * _NS
_CHUNK = N // _NW
_RB = 64
_IDS = 1024
_TGRP = 16
_WIN = 128
_NV = F // 16


def _zero_window(win_v):
    def body(r, _):
        win_v[pl.ds(16 * r, 16)] = jnp.zeros((16,), jnp.float32)
        return 0
    lax.fori_loop(0, _WIN * _NV, body, 0)


def _segmax_body(feat_hbm, p2v_hbm, out_hbm,
                 bnd_v, rows_v, tid_v, win_v, ids_v, acc_v, done_s):
    wid = lax.axis_index("s") * _NC + lax.axis_index("c")
    start = wid * _CHUNK
    end = start + _CHUNK

    boff = pl.multiple_of(jnp.where(wid > 0, start - 16, 0), 16)
    pltpu.sync_copy(p2v_hbm.at[pl.ds(boff, 16)], bnd_v)
    prev = jnp.where(wid > 0, bnd_v[pl.ds(0, 16)][15], jnp.int32(-1))

    pltpu.sync_copy(p2v_hbm.at[pl.ds(pl.multiple_of(start, _CHUNK), _CHUNK)],
                    ids_v.at[pl.ds(0, _CHUNK)])

    _zero_window(win_v)
    done_s[0] = jnp.int32(0)

    neg_inf = jnp.full((16,), -jnp.inf, jnp.float32)

    def flush_windows(v, wb):
        def w_body(k, w):
            pltpu.sync_copy(
                win_v, out_hbm.at[pl.ds(pl.multiple_of(w * F, 256), _WIN * F)])
            _zero_window(win_v)
            return w + _WIN

        return lax.fori_loop(0, (v - wb) // _WIN, w_body, wb)

    def store_seg(v, acc, wb):
        wb = flush_windows(v, wb)
        idx = v - wb
        for i in range(_NV):
            win_v[pl.ds(pl.multiple_of(idx * F + 16 * i, 16), 16)] = acc[i]
        return wb

    def row_step(local_r, rid, carry):
        cur, wb = carry[0], carry[1]
        acc = carry[2:]
        row = [rows_v[local_r, pl.ds(16 * i, 16)] for i in range(_NV)]
        changed = rid != cur

        wb = lax.cond(changed & (cur > prev),
                      lambda w: store_seg(cur, acc, w),
                      lambda w: w, wb)

        acc = [jnp.where(changed, row[i], jnp.maximum(acc[i], row[i]))
               for i in range(_NV)]
        return (rid, wb) + tuple(acc)

    def block_body(b, carry):
        pltpu.sync_copy(
            feat_hbm.at[pl.ds(pl.multiple_of(start + b * _RB, _RB), _RB), :],
            rows_v)

        def inner(j, c):
            rid = ids_v[pl.ds(b * _RB + j, 16)][0]
            return row_step(j, rid, c)

        return lax.fori_loop(0, _RB, inner, carry)

    carry0 = (prev, prev + 1) + tuple(neg_inf for _ in range(_NV))
    carry = lax.fori_loop(0, _CHUNK // _RB, block_body, carry0)
    cur, wbase = carry[0], carry[1]
    acc = list(carry[2:])

    for i in range(_NV):
        acc_v[pl.ds(16 * i, 16)] = acc[i]

    def t_block(g, b):
        rpa = pl.multiple_of(end + (g * _TGRP + b) * _RB, _RB)
        pltpu.sync_copy(p2v_hbm.at[pl.ds(rpa, _RB)], tid_v.at[pl.ds(0, _RB)])
        pltpu.sync_copy(feat_hbm.at[pl.ds(rpa, _RB), :], rows_v)

        def inner(j, c):
            tj = tid_v[pl.ds(j, 16)][0]

            @pl.when((done_s[0] == 0) & (tj != cur))
            def _():
                done_s[0] = jnp.int32(1)

            @pl.when((done_s[0] == 0) & (tj == cur))
            def _():
                for i in range(_NV):
                    row = rows_v[j, pl.ds(16 * i, 16)]
                    a = acc_v[pl.ds(16 * i, 16)]
                    acc_v[pl.ds(16 * i, 16)] = jnp.maximum(a, row)

            return c

        return lax.fori_loop(0, _RB, inner, 0)

    def t_group(g, c):
        @pl.when(done_s[0] == 0)
        def _():
            def blk(b, cc):
                @pl.when(done_s[0] == 0)
                def _():
                    t_block(g, b)

                return cc

            lax.fori_loop(0, _TGRP, blk, 0)

        return c

    n_tail_groups = (N - end) // (_TGRP * _RB)
    lax.fori_loop(0, n_tail_groups, t_group, 0)
    acc = [acc_v[pl.ds(16 * i, 16)] for i in range(_NV)]

    wbase = lax.cond(cur > prev,
                     lambda w: store_seg(cur, acc, w),
                     lambda w: w, wbase)

    hi = jnp.where(wid == _NW - 1, jnp.int32(M - 1), cur)
    hi = jnp.maximum(hi, prev)

    def g_body(k, w):
        pltpu.sync_copy(
            win_v, out_hbm.at[pl.ds(pl.multiple_of(w * F, 256), _WIN * F)])
        _zero_window(win_v)
        return w + _WIN

    wbase = lax.fori_loop(0, (hi - wbase) // _WIN, g_body, wbase)

    def s_body(k, w):
        pltpu.sync_copy(
            win_v.at[pl.ds(pl.multiple_of(k * F, 256), F)],
            out_hbm.at[pl.ds(pl.multiple_of((w + k) * F, 256), F)])
        return w

    lax.fori_loop(0, hi + 1 - wbase, s_body, wbase)


def _run_segmax(feat, p2v):
    mesh = plsc.VectorSubcoreMesh(core_axis_name="c", subcore_axis_name="s",
                                  num_cores=_NC, num_subcores=_NS)
    k = pl.kernel(
        _segmax_body,
        out_type=jax.ShapeDtypeStruct((M * F,), jnp.float32),
        mesh=mesh,
        scratch_types=[
            pltpu.VMEM((16,), jnp.int32),
            pltpu.VMEM((_RB, F), jnp.float32),
            pltpu.VMEM((_RB + 16,), jnp.int32),
            pltpu.VMEM((_WIN * F,), jnp.float32),
            pltpu.VMEM((_CHUNK + 16,), jnp.int32),
            pltpu.VMEM((F,), jnp.float32),
            pltpu.SMEM((1,), jnp.int32),
        ],
    )
    return k(feat, p2v).reshape(M, F)


def kernel(x, p2v, g0, be0, W1, b1, g1, be1, W2, b2, g2, be2, W3, b3, g3, be3,
           W4, b4, Wc, bc):
    s1_, q1_ = _xstats(x)
    mx = s1_[0] / N
    C = q1_ / N - jnp.outer(mx, mx)

    v0 = jnp.diagonal(C)
    s0 = g0 * lax.rsqrt(v0 + _EPS)
    t0 = be0 - mx * s0
    W1f = s0[:, None] * W1
    b1f = b1 + t0 @ W1

    m1 = mx @ W1f + b1f
    v1 = jnp.einsum("aj,aj->j", W1f, C @ W1f)
    sc1 = g1 * lax.rsqrt(v1 + _EPS)
    tc1 = be1 - m1 * sc1

    ws = {
        "W1f": W1f, "b1f": b1f.reshape(1, -1),
        "s1": sc1.reshape(1, -1), "t1": tc1.reshape(1, -1),
        "W2": W2, "b2": b2.reshape(1, -1),
        "W3": W3, "b3": b3.reshape(1, -1),
        "W4": W4, "b4": b4.reshape(1, -1),
    }

    st2 = _run_stats(x, ws, 3, 128)[0]
    m2 = st2[0] / N
    v2 = st2[1] / N - m2 * m2
    sc2 = g2 * lax.rsqrt(v2 + _EPS)
    ws["s2"] = sc2.reshape(1, -1)
    ws["t2"] = (be2 - m2 * sc2).reshape(1, -1)

    st3 = _run_stats(x, ws, 4, 256)[0]
    m3 = st3[0] / N
    v3 = st3[1] / N - m3 * m3
    sc3 = g3 * lax.rsqrt(v3 + _EPS)
    ws["s3"] = sc3.reshape(1, -1)
    ws["t3"] = (be3 - m3 * sc3).reshape(1, -1)

    feat = _run_feat(x, ws)
    smax = _run_segmax(feat, p2v)
    cfeat = _run_compress(smax, Wc, bc)
    return (cfeat, feat)


# speedup vs baseline: 2.1331x; 2.1331x over previous
import functools

import jax
import jax.numpy as jnp
from jax import lax
from jax.experimental import pallas as pl
from jax.experimental.pallas import tpu as pltpu
from jax.experimental.pallas import tpu_sc as plsc

N = 262144
CIN = 9
M = 40000
F = 256
COMPR = 16

_PREC = lax.Precision.HIGHEST
_PREC_STATS = lax.Precision.DEFAULT
_EPS = 1e-5


def _dot1(a, b):
    return lax.dot_general(a, b, (((1,), (0,)), ((), ())),
                           preferred_element_type=jnp.float32,
                           precision=_PREC_STATS)


def _dot3(a, b):
    ah = a.astype(jnp.bfloat16)
    al = (a - ah.astype(jnp.float32)).astype(jnp.bfloat16)
    bh = b.astype(jnp.bfloat16)
    bl = (b - bh.astype(jnp.float32)).astype(jnp.bfloat16)

    def d(u, v):
        return lax.dot_general(u, v, (((1,), (0,)), ((), ())),
                               preferred_element_type=jnp.float32)

    return d(ah, bh) + (d(ah, bl) + d(al, bh))


_BX = 2048


def _lrelu(x):
    return jnp.where(x >= 0, x, 0.01 * x)


def _xstats_kernel(x_ref, s_ref, q_ref):
    i = pl.program_id(0)

    @pl.when(i == 0)
    def _():
        s_ref[...] = jnp.zeros_like(s_ref)
        q_ref[...] = jnp.zeros_like(q_ref)

    xb = x_ref[...]
    s_ref[...] += jnp.sum(xb, axis=0, keepdims=True)
    q_ref[...] += lax.dot_general(
        xb, xb, (((0,), (0,)), ((), ())),
        preferred_element_type=jnp.float32, precision=_PREC)


def _xstats(x):
    return pl.pallas_call(
        _xstats_kernel,
        grid=(N // _BX,),
        in_specs=[pl.BlockSpec((_BX, CIN), lambda i: (i, 0))],
        out_specs=[
            pl.BlockSpec((1, CIN), lambda i: (0, 0)),
            pl.BlockSpec((CIN, CIN), lambda i: (0, 0)),
        ],
        out_shape=[
            jax.ShapeDtypeStruct((1, CIN), jnp.float32),
            jax.ShapeDtypeStruct((CIN, CIN), jnp.float32),
        ],
    )(x)


def _fwd(x, ws, depth, dotf):
    h = dotf(x, ws["W1f"]) + ws["b1f"]
    if depth == 2:
        return h
    h = _lrelu(h * ws["s1"] + ws["t1"])
    h = dotf(h, ws["W2"]) + ws["b2"]
    if depth == 3:
        return h
    h = _lrelu(h * ws["s2"] + ws["t2"])
    h = dotf(h, ws["W3"]) + ws["b3"]
    if depth == 4:
        return h
    h = _lrelu(h * ws["s3"] + ws["t3"])
    h = dotf(h, ws["W4"]) + ws["b4"]
    return h


_STAT_KEYS = {
    3: ("W1f", "b1f", "s1", "t1", "W2", "b2"),
    4: ("W1f", "b1f", "s1", "t1", "W2", "b2", "s2", "t2", "W3", "b3"),
    5: ("W1f", "b1f", "s1", "t1", "W2", "b2", "s2", "t2", "W3", "b3",
        "s3", "t3", "W4", "b4"),
}


def _stats_kernel(depth, keys, x_ref, *rest):
    w_refs, (o_ref,) = rest[:len(keys)], rest[len(keys):]
    i = pl.program_id(0)

    @pl.when(i == 0)
    def _():
        o_ref[...] = jnp.zeros_like(o_ref)

    ws = {k: r[...] for k, r in zip(keys, w_refs)}
    a = _fwd(x_ref[...], ws, depth, _dot1)
    o_ref[0:1, :] += jnp.sum(a, axis=0, keepdims=True)
    o_ref[1:2, :] += jnp.sum(a * a, axis=0, keepdims=True)


def _run_stats(x, ws, depth, width):
    keys = _STAT_KEYS[depth]
    return pl.pallas_call(
        functools.partial(_stats_kernel, depth, keys),
        grid=(N // _BX,),
        in_specs=[pl.BlockSpec((_BX, CIN), lambda i: (i, 0))] + [
            pl.BlockSpec(ws[k].shape, lambda i: (0, 0)) for k in keys],
        out_specs=[pl.BlockSpec((2, width), lambda i: (0, 0))],
        out_shape=[jax.ShapeDtypeStruct((2, width), jnp.float32)],
    )(x, *[ws[k] for k in keys])


def _feat_kernel(keys, x_ref, *rest):
    w_refs, (o_ref,) = rest[:len(keys)], rest[len(keys):]
    ws = {k: r[...] for k, r in zip(keys, w_refs)}
    o_ref[...] = _fwd(x_ref[...], ws, 5, _dot3)


def _run_feat(x, ws):
    keys = _STAT_KEYS[5]
    return pl.pallas_call(
        functools.partial(_feat_kernel, keys),
        grid=(N // _BX,),
        in_specs=[pl.BlockSpec((_BX, CIN), lambda i: (i, 0))] + [
            pl.BlockSpec(ws[k].shape, lambda i: (0, 0)) for k in keys],
        out_specs=[pl.BlockSpec((_BX, F), lambda i: (i, 0))],
        out_shape=[jax.ShapeDtypeStruct((N, F), jnp.float32)],
    )(x, *[ws[k] for k in keys])[0]


_BM = 2000


def _compress_kernel(s_ref, w_ref, b_ref, o_ref):
    o_ref[...] = _lrelu(_dot3(s_ref[...], w_ref[...]) + b_ref[...])


def _run_compress(smax, Wc, bc):
    return pl.pallas_call(
        _compress_kernel,
        grid=(M // _BM,),
        in_specs=[
            pl.BlockSpec((_BM, F), lambda i: (i, 0)),
            pl.BlockSpec((F, COMPR), lambda i: (0, 0)),
            pl.BlockSpec((1, COMPR), lambda i: (0, 0)),
        ],
        out_specs=[pl.BlockSpec((_BM, COMPR), lambda i: (i, 0))],
        out_shape=[jax.ShapeDtypeStruct((M, COMPR), jnp.float32)],
    )(smax, Wc, bc.reshape(1, COMPR))[0]


_NC = 2
_NS = 16
_NW = _NC * _NS
_CHUNK = N // _NW
_RB = 128
_IDS = 1024
_TGRP = 16
_WIN = 128
_NV = F // 16


def _zero_window(win_v):
    def body(r, _):
        win_v[pl.ds(16 * r, 16)] = jnp.zeros((16,), jnp.float32)
        return 0
    lax.fori_loop(0, _WIN * _NV, body, 0)


def _segmax_body(feat_hbm, p2v_hbm, out_hbm,
                 bnd_v, rows0_v, rows1_v, tid_v, win_v, ids_v, acc_v, done_s,
                 sem0, sem1):
    wid = lax.axis_index("s") * _NC + lax.axis_index("c")
    start = wid * _CHUNK
    end = start + _CHUNK

    boff = pl.multiple_of(jnp.where(wid > 0, start - 16, 0), 16)
    pltpu.sync_copy(p2v_hbm.at[pl.ds(boff, 16)], bnd_v)
    prev = jnp.where(wid > 0, bnd_v[pl.ds(0, 16)][15], jnp.int32(-1))

    pltpu.sync_copy(p2v_hbm.at[pl.ds(pl.multiple_of(start, _CHUNK), _CHUNK)],
                    ids_v.at[pl.ds(0, _CHUNK)])

    _zero_window(win_v)
    done_s[0] = jnp.int32(0)

    neg_inf = jnp.full((16,), -jnp.inf, jnp.float32)

    def flush_windows(v, wb):
        def w_body(k, w):
            pltpu.sync_copy(
                win_v, out_hbm.at[pl.ds(pl.multiple_of(w * F, 256), _WIN * F)])
            _zero_window(win_v)
            return w + _WIN

        return lax.fori_loop(0, (v - wb) // _WIN, w_body, wb)

    def store_seg(v, acc, wb):
        wb = flush_windows(v, wb)
        idx = v - wb
        for i in range(_NV):
            win_v[pl.ds(pl.multiple_of(idx * F + 16 * i, 16), 16)] = acc[i]
        return wb

    def row_step(buf, local_r, rid, carry):
        cur, wb = carry[0], carry[1]
        acc = carry[2:]
        row = [buf[local_r, pl.ds(16 * i, 16)] for i in range(_NV)]
        changed = rid != cur

        wb = lax.cond(changed & (cur > prev),
                      lambda w: store_seg(cur, acc, w),
                      lambda w: w, wb)

        acc = [jnp.where(changed, row[i], jnp.maximum(acc[i], row[i]))
               for i in range(_NV)]
        return (rid, wb) + tuple(acc)

    nb = _CHUNK // _RB

    def feat_block(b):
        return feat_hbm.at[pl.ds(pl.multiple_of(start + b * _RB, _RB), _RB), :]

    def process(b, buf, carry):
        def sub(q, c):
            idv = ids_v[pl.ds(b * _RB + q * 16, 16)]
            for k in range(16):
                c = row_step(buf, q * 16 + k, idv[k], c)
            return c

        return lax.fori_loop(0, _RB // 16, sub, carry)

    pltpu.async_copy(feat_block(0), rows0_v, sem0)

    def pair(g, carry):
        b0 = 2 * g
        b1 = 2 * g + 1
        pltpu.async_copy(feat_block(b1), rows1_v, sem1)
        pltpu.make_async_copy(feat_block(0), rows0_v, sem0).wait()
        carry = process(b0, rows0_v, carry)
        pltpu.async_copy(feat_block(jnp.minimum(b0 + 2, nb - 1)), rows0_v,
                         sem0)
        pltpu.make_async_copy(feat_block(0), rows1_v, sem1).wait()
        return process(b1, rows1_v, carry)

    carry0 = (prev, prev + 1) + tuple(neg_inf for _ in range(_NV))
    carry = lax.fori_loop(0, nb // 2, pair, carry0)
    pltpu.make_async_copy(feat_block(0), rows0_v, sem0).wait()
    cur, wbase = carry[0], carry[1]
    acc = list(carry[2:])

    for i in range(_NV):
        acc_v[pl.ds(16 * i, 16)] = acc[i]

    def t_block(g, b):
        rpa = pl.multiple_of(end + (g * _TGRP + b) * _RB, _RB)
        pltpu.sync_copy(p2v_hbm.at[pl.ds(rpa, _RB)], tid_v.at[pl.ds(0, _RB)])
        pltpu.sync_copy(feat_hbm.at[pl.ds(rpa, _RB), :], rows0_v)

        def inner(j, c):
            tj = tid_v[pl.ds(j, 16)][0]

            @pl.when((done_s[0] == 0) & (tj != cur))
            def _():
                done_s[0] = jnp.int32(1)

            @pl.when((done_s[0] == 0) & (tj == cur))
            def _():
                for i in range(_NV):
                    row = rows0_v[j, pl.ds(16 * i, 16)]
                    a = acc_v[pl.ds(16 * i, 16)]
                    acc_v[pl.ds(16 * i, 16)] = jnp.maximum(a, row)

            return c

        return lax.fori_loop(0, _RB, inner, 0)

    def t_group(g, c):
        @pl.when(done_s[0] == 0)
        def _():
            def blk(b, cc):
                @pl.when(done_s[0] == 0)
                def _():
                    t_block(g, b)

                return cc

            lax.fori_loop(0, _TGRP, blk, 0)

        return c

    n_tail_groups = (N - end) // (_TGRP * _RB)
    lax.fori_loop(0, n_tail_groups, t_group, 0)
    acc = [acc_v[pl.ds(16 * i, 16)] for i in range(_NV)]

    wbase = lax.cond(cur > prev,
                     lambda w: store_seg(cur, acc, w),
                     lambda w: w, wbase)

    hi = jnp.where(wid == _NW - 1, jnp.int32(M - 1), cur)
    hi = jnp.maximum(hi, prev)

    def g_body(k, w):
        pltpu.sync_copy(
            win_v, out_hbm.at[pl.ds(pl.multiple_of(w * F, 256), _WIN * F)])
        _zero_window(win_v)
        return w + _WIN

    wbase = lax.fori_loop(0, (hi - wbase) // _WIN, g_body, wbase)

    def s_body(k, w):
        pltpu.sync_copy(
            win_v.at[pl.ds(pl.multiple_of(k * F, 256), F)],
            out_hbm.at[pl.ds(pl.multiple_of((w + k) * F, 256), F)])
        return w

    lax.fori_loop(0, hi + 1 - wbase, s_body, wbase)


def _run_segmax(feat, p2v):
    mesh = plsc.VectorSubcoreMesh(core_axis_name="c", subcore_axis_name="s",
                                  num_cores=_NC, num_subcores=_NS)
    k = pl.kernel(
        _segmax_body,
        out_type=jax.ShapeDtypeStruct((M * F,), jnp.float32),
        mesh=mesh,
        scratch_types=[
            pltpu.VMEM((16,), jnp.int32),
            pltpu.VMEM((_RB, F), jnp.float32),
            pltpu.VMEM((_RB, F), jnp.float32),
            pltpu.VMEM((_RB + 16,), jnp.int32),
            pltpu.VMEM((_WIN * F,), jnp.float32),
            pltpu.VMEM((_CHUNK + 16,), jnp.int32),
            pltpu.VMEM((F,), jnp.float32),
            pltpu.SMEM((1,), jnp.int32),
            pltpu.SemaphoreType.DMA,
            pltpu.SemaphoreType.DMA,
        ],
    )
    return k(feat, p2v).reshape(M, F)


def kernel(x, p2v, g0, be0, W1, b1, g1, be1, W2, b2, g2, be2, W3, b3, g3, be3,
           W4, b4, Wc, bc):
    s1_, q1_ = _xstats(x)
    mx = s1_[0] / N
    C = q1_ / N - jnp.outer(mx, mx)

    v0 = jnp.diagonal(C)
    s0 = g0 * lax.rsqrt(v0 + _EPS)
    t0 = be0 - mx * s0
    W1f = s0[:, None] * W1
    b1f = b1 + t0 @ W1

    m1 = mx @ W1f + b1f
    v1 = jnp.einsum("aj,aj->j", W1f, C @ W1f)
    sc1 = g1 * lax.rsqrt(v1 + _EPS)
    tc1 = be1 - m1 * sc1

    ws = {
        "W1f": W1f, "b1f": b1f.reshape(1, -1),
        "s1": sc1.reshape(1, -1), "t1": tc1.reshape(1, -1),
        "W2": W2, "b2": b2.reshape(1, -1),
        "W3": W3, "b3": b3.reshape(1, -1),
        "W4": W4, "b4": b4.reshape(1, -1),
    }

    st2 = _run_stats(x, ws, 3, 128)[0]
    m2 = st2[0] / N
    v2 = st2[1] / N - m2 * m2
    sc2 = g2 * lax.rsqrt(v2 + _EPS)
    ws["s2"] = sc2.reshape(1, -1)
    ws["t2"] = (be2 - m2 * sc2).reshape(1, -1)

    st3 = _run_stats(x, ws, 4, 256)[0]
    m3 = st3[0] / N
    v3 = st3[1] / N - m3 * m3
    sc3 = g3 * lax.rsqrt(v3 + _EPS)
    ws["s3"] = sc3.reshape(1, -1)
    ws["t3"] = (be3 - m3 * sc3).reshape(1, -1)

    feat = _run_feat(x, ws)
    smax = _run_segmax(feat, p2v)
    cfeat = _run_compress(smax, Wc, bc)
    return (cfeat, feat)
